# baseline (speedup 1.0000x reference)
import functools

import jax
import jax.numpy as jnp
from jax import lax
from jax.experimental import pallas as pl
from jax.experimental.pallas import tpu as pltpu
from jax.experimental.pallas import tpu_sc as plsc

VOCAB = 1000000
D = 64
T = 200
B = 1024

NC = 2
NS = 16
NW = NC * NS
ROWS = B * T
ROWS_W = ROWS // NW
SEQ_CHUNK = 4
C = SEQ_CHUNK * T
NCHUNK = ROWS_W // C
GSUB = 128
NLOAD = (C + GSUB - 1) // GSUB


def _make_kernel():
    mesh = plsc.VectorSubcoreMesh(core_axis_name="c", subcore_axis_name="s")

    @functools.partial(
        pl.kernel,
        out_type=jax.ShapeDtypeStruct((ROWS, D), jnp.float32),
        mesh=mesh,
        scratch_types=[
            pltpu.VMEM((T, D), jnp.float32),
            pltpu.VMEM((C,), jnp.int32),
            pltpu.VMEM((C,), jnp.int32),
            pltpu.VMEM((C, D), jnp.float32),
            pltpu.VMEM((C, D), jnp.float32),
            pltpu.SemaphoreType.DMA,
            pltpu.SemaphoreType.DMA,
            pltpu.SemaphoreType.DMA,
            pltpu.SemaphoreType.DMA,
        ],
        compiler_params=pltpu.CompilerParams(use_tc_tiling_on_sc=False),
    )
    def k(x_hbm, emb_hbm, pos_hbm, out_hbm, pos_v, idx0, idx1, rows0, rows1,
          gsem0, gsem1, ssem0, ssem1):
        idx_v = (idx0, idx1)
        rows_v = (rows0, rows1)
        gsem = (gsem0, gsem1)
        ssem = (ssem0, ssem1)
        wid = lax.axis_index("s") * NC + lax.axis_index("c")
        base = wid * ROWS_W

        pltpu.sync_copy(pos_hbm, pos_v)

        def start_gathers(g, bi):
            pltpu.sync_copy(x_hbm.at[pl.ds(base + g * C, C)], idx_v[bi])
            handles = []
            for j in range(NLOAD):
                n = min(GSUB, C - j * GSUB)
                handles.append(pltpu.async_copy(
                    emb_hbm.at[idx_v[bi].at[pl.ds(j * GSUB, n)]],
                    rows_v[bi].at[pl.ds(j * GSUB, n)],
                    gsem[bi]))
            return handles

        def add_pos(bi):
            rows = rows_v[bi]

            def body(t, carry):
                pv = [pos_v[t, pl.ds(j * 16, 16)] for j in range(4)]
                for s in range(SEQ_CHUNK):
                    for j in range(4):
                        plsc.addupdate(rows.at[s * T + t, pl.ds(j * 16, 16)],
                                       pv[j])
                return carry

            lax.fori_loop(0, T, body, 0)

        pending_g = [None, None]
        pending_s = [None, None]
        pending_g[0] = start_gathers(0, 0)
        for g in range(NCHUNK):
            bi = g % 2
            nb = (g + 1) % 2
            if g + 1 < NCHUNK:
                if pending_s[nb] is not None:
                    pending_s[nb].wait()
                    pending_s[nb] = None
                pending_g[nb] = start_gathers(g + 1, nb)
            for h in pending_g[bi]:
                h.wait()
            pending_g[bi] = None
            add_pos(bi)
            pending_s[bi] = pltpu.async_copy(
                rows_v[bi], out_hbm.at[pl.ds(base + g * C, C)], ssem[bi])
        for s in pending_s:
            if s is not None:
                s.wait()

    return k


_kernel_call = _make_kernel()


@jax.jit
def kernel(x, emb, posenc):
    flat = _kernel_call(x.reshape(-1), emb, posenc)
    return flat.reshape(B, T, D)
